# Initial kernel scaffold; baseline (speedup 1.0000x reference)
import jax
import jax.numpy as jnp
from jax.experimental import pallas as pl


def kernel(x, edge_index, W, att_src, att_dst, bias):
    raise NotImplementedError("write your pallas kernel here")


# speedup vs baseline: 65.8853x; 65.8853x over previous
import functools

import jax
import jax.numpy as jnp
from jax import lax
from jax.experimental import pallas as pl
from jax.experimental.pallas import tpu as pltpu
from jax.experimental.pallas import tpu_sc as plsc

N_NODES = 10000
IN_DIM = 128
HEADS = 4
OUT_DIM = 32
HD = HEADS * OUT_DIM
NEG_SLOPE = 0.2
NPAD = 10112
K_EDGE = 128


def _proj_body(x_ref, w_ref, asrc_ref, adst_ref, xp_ref, ast_ref, adt_ref):
    xp = jnp.dot(x_ref[...], w_ref[...], preferred_element_type=jnp.float32)
    xp_ref[...] = xp
    col = lax.broadcasted_iota(jnp.int32, (HD, 16), 0) // OUT_DIM
    k16 = lax.broadcasted_iota(jnp.int32, (HD, 16), 1)
    m2 = jnp.where((k16 < HEADS) & (col == k16), 1.0, 0.0)
    ast_ref[...] = jnp.dot(xp * asrc_ref[...], m2, preferred_element_type=jnp.float32)
    adt_ref[...] = jnp.dot(xp * adst_ref[...], m2, preferred_element_type=jnp.float32)


def _project(x, w, att_src, att_dst):
    rows = 2000
    grid = N_NODES // rows
    return pl.pallas_call(
        _proj_body,
        grid=(grid,),
        in_specs=[
            pl.BlockSpec((rows, IN_DIM), lambda i: (i, 0)),
            pl.BlockSpec((IN_DIM, HD), lambda i: (0, 0)),
            pl.BlockSpec((1, HD), lambda i: (0, 0)),
            pl.BlockSpec((1, HD), lambda i: (0, 0)),
        ],
        out_specs=[
            pl.BlockSpec((rows, HD), lambda i: (i, 0)),
            pl.BlockSpec((rows, 16), lambda i: (i, 0)),
            pl.BlockSpec((rows, 16), lambda i: (i, 0)),
        ],
        out_shape=[
            jax.ShapeDtypeStruct((N_NODES, HD), jnp.float32),
            jax.ShapeDtypeStruct((N_NODES, 16), jnp.float32),
            jax.ShapeDtypeStruct((N_NODES, 16), jnp.float32),
        ],
    )(x, w, att_src.reshape(1, HD), att_dst.reshape(1, HD))


def _sc_info():
    try:
        info = plsc.get_sparse_core_info()
        return info.num_cores, info.num_subcores
    except Exception:
        return 2, 16


def _edge_body(num_rounds, num_cores, num_subcores,
               xp_hbm, ast_hbm, adt_hbm, src_hbm, dst_hbm, zn_hbm, zd_hbm,
               pnum_hbm, pden_hbm,
               acc_n, acc_d, sidx, didx, asr, adr, xrows, wbuf,
               sem_a, sem_b, sem_x, sem_s):
    c = lax.axis_index("c")
    s = lax.axis_index("s")
    wid = s * num_cores + c
    rows_per = NPAD // num_subcores
    r0 = s * rows_per
    pltpu.sync_copy(zn_hbm.at[pl.ds(r0, rows_per)], acc_n.at[pl.ds(r0, rows_per)])
    pltpu.sync_copy(zd_hbm.at[pl.ds(r0, rows_per)], acc_d.at[pl.ds(r0, rows_per)])
    plsc.subcore_barrier()

    def round_body(g, carry):
        pltpu.sync_copy(src_hbm.at[wid, g], sidx)
        pltpu.sync_copy(dst_hbm.at[wid, g], didx)
        cp_a = pltpu.async_copy(ast_hbm.at[sidx], asr, sem_a)
        cp_b = pltpu.async_copy(adt_hbm.at[didx], adr, sem_b)
        cp_x = pltpu.async_copy(xp_hbm.at[sidx], xrows, sem_x)
        cp_a.wait()
        cp_b.wait()
        cp_x.wait()

        def edge_body(e, carry2):
            att = asr[e, :] + adr[e, :]
            att = jnp.where(att >= 0.0, att, att * NEG_SLOPE)
            wrow = jnp.exp(att)
            wbuf[e, :] = wrow
            for h in range(HEADS):
                wv = jnp.full((16,), wrow[h], jnp.float32)
                for j2 in range(2):
                    off = (h * 2 + j2) * 16
                    xrows[e, pl.ds(off, 16)] = xrows[e, pl.ds(off, 16)] * wv
            return carry2

        lax.fori_loop(0, K_EDGE, edge_body, 0)
        pltpu.async_copy(xrows, acc_n.at[didx], sem_s, add=True).wait()
        pltpu.async_copy(wbuf, acc_d.at[didx], sem_s, add=True).wait()
        return carry

    lax.fori_loop(0, num_rounds, round_body, 0)
    plsc.subcore_barrier()
    pltpu.sync_copy(acc_n.at[pl.ds(r0, rows_per)], pnum_hbm.at[c, pl.ds(r0, rows_per)])
    pltpu.sync_copy(acc_d.at[pl.ds(r0, rows_per)], pden_hbm.at[c, pl.ds(r0, rows_per)])


def _edge_pass(xp, ast_pad, adt_pad, src3, dst3, num_rounds, num_cores, num_subcores):
    mesh = plsc.VectorSubcoreMesh(core_axis_name="c", subcore_axis_name="s")
    zn = jnp.zeros((NPAD, HD), jnp.float32)
    zd = jnp.zeros((NPAD, 16), jnp.float32)
    body = functools.partial(_edge_body, num_rounds, num_cores, num_subcores)
    return pl.kernel(
        body,
        out_type=[
            jax.ShapeDtypeStruct((num_cores, NPAD, HD), jnp.float32),
            jax.ShapeDtypeStruct((num_cores, NPAD, 16), jnp.float32),
        ],
        mesh=mesh,
        compiler_params=pltpu.CompilerParams(use_tc_tiling_on_sc=False),
        scratch_types=[
            pltpu.VMEM_SHARED((NPAD, HD), jnp.float32),
            pltpu.VMEM_SHARED((NPAD, 16), jnp.float32),
            pltpu.VMEM((K_EDGE,), jnp.int32),
            pltpu.VMEM((K_EDGE,), jnp.int32),
            pltpu.VMEM((K_EDGE, 16), jnp.float32),
            pltpu.VMEM((K_EDGE, 16), jnp.float32),
            pltpu.VMEM((K_EDGE, HD), jnp.float32),
            pltpu.VMEM((K_EDGE, 16), jnp.float32),
            pltpu.SemaphoreType.DMA,
            pltpu.SemaphoreType.DMA,
            pltpu.SemaphoreType.DMA,
            pltpu.SemaphoreType.DMA,
        ],
    )(xp, ast_pad, adt_pad, src3, dst3, zn, zd)


def _combine_body(pnum_ref, pden_ref, bias_ref, out_ref):
    pn = pnum_ref[...]
    pd = pden_ref[...]
    num = pn[0] + pn[1]
    den = pd[0] + pd[1]
    r16 = lax.broadcasted_iota(jnp.int32, (16, HD), 0)
    c16 = lax.broadcasted_iota(jnp.int32, (16, HD), 1) // OUT_DIM
    m16 = jnp.where(r16 == c16, 1.0, 0.0)
    den_b = jnp.dot(den, m16, preferred_element_type=jnp.float32)
    out_ref[...] = num / den_b + bias_ref[...]


def _combine(pnum, pden, bias, num_cores):
    rows = 2000
    grid = N_NODES // rows
    return pl.pallas_call(
        _combine_body,
        grid=(grid,),
        in_specs=[
            pl.BlockSpec((num_cores, rows, HD), lambda i: (0, i, 0)),
            pl.BlockSpec((num_cores, rows, 16), lambda i: (0, i, 0)),
            pl.BlockSpec((1, HD), lambda i: (0, 0)),
        ],
        out_specs=pl.BlockSpec((rows, HD), lambda i: (i, 0)),
        out_shape=jax.ShapeDtypeStruct((N_NODES, HD), jnp.float32),
    )(pnum, pden, bias.reshape(1, HD))


def kernel(x, edge_index, W, att_src, att_dst, bias):
    num_cores, num_subcores = _sc_info()
    nw = num_cores * num_subcores
    xp, ast, adt = _project(x, W, att_src, att_dst)
    ast_pad = jnp.pad(ast, ((0, NPAD - N_NODES), (0, 0)))
    adt_pad = jnp.pad(adt, ((0, NPAD - N_NODES), (0, 0)))

    self_loops = jnp.arange(N_NODES, dtype=jnp.int32)
    src = jnp.concatenate([edge_index[0], self_loops])
    dst = jnp.concatenate([edge_index[1], self_loops])
    e_tot = src.shape[0]
    chunk = nw * K_EDGE
    num_rounds = -(-e_tot // chunk)
    e_pad = num_rounds * chunk - e_tot
    src = jnp.pad(src, (0, e_pad)).reshape(nw, num_rounds, K_EDGE)
    dst = jnp.pad(dst, (0, e_pad), constant_values=N_NODES).reshape(nw, num_rounds, K_EDGE)

    pnum, pden = _edge_pass(xp, ast_pad, adt_pad, src, dst, num_rounds, num_cores, num_subcores)
    return _combine(pnum, pden, bias, num_cores)


# speedup vs baseline: 78.9657x; 1.1985x over previous
import functools

import jax
import jax.numpy as jnp
from jax import lax
from jax.experimental import pallas as pl
from jax.experimental.pallas import tpu as pltpu
from jax.experimental.pallas import tpu_sc as plsc

N_NODES = 10000
IN_DIM = 128
HEADS = 4
OUT_DIM = 32
HD = HEADS * OUT_DIM
NEG_SLOPE = 0.2
NPAD = 10112
K_EDGE = 64


def _proj_body(x_ref, w_ref, asrc_ref, adst_ref, xp_ref, ast_ref, adt_ref):
    xp = jnp.dot(x_ref[...], w_ref[...], preferred_element_type=jnp.float32)
    xp_ref[...] = xp
    col = lax.broadcasted_iota(jnp.int32, (HD, 16), 0) // OUT_DIM
    k16 = lax.broadcasted_iota(jnp.int32, (HD, 16), 1)
    m2 = jnp.where((k16 < HEADS) & (col == k16), 1.0, 0.0)
    ast_ref[...] = jnp.dot(xp * asrc_ref[...], m2, preferred_element_type=jnp.float32)
    adt_ref[...] = jnp.dot(xp * adst_ref[...], m2, preferred_element_type=jnp.float32)


def _project(x, w, att_src, att_dst):
    rows = 2000
    grid = N_NODES // rows
    return pl.pallas_call(
        _proj_body,
        grid=(grid,),
        in_specs=[
            pl.BlockSpec((rows, IN_DIM), lambda i: (i, 0)),
            pl.BlockSpec((IN_DIM, HD), lambda i: (0, 0)),
            pl.BlockSpec((1, HD), lambda i: (0, 0)),
            pl.BlockSpec((1, HD), lambda i: (0, 0)),
        ],
        out_specs=[
            pl.BlockSpec((rows, HD), lambda i: (i, 0)),
            pl.BlockSpec((rows, 16), lambda i: (i, 0)),
            pl.BlockSpec((rows, 16), lambda i: (i, 0)),
        ],
        out_shape=[
            jax.ShapeDtypeStruct((N_NODES, HD), jnp.float32),
            jax.ShapeDtypeStruct((N_NODES, 16), jnp.float32),
            jax.ShapeDtypeStruct((N_NODES, 16), jnp.float32),
        ],
    )(x, w, att_src.reshape(1, HD), att_dst.reshape(1, HD))


def _sc_info():
    try:
        info = plsc.get_sparse_core_info()
        return info.num_cores, info.num_subcores
    except Exception:
        return 2, 16


def _edge_body(num_rounds, num_cores, num_subcores,
               xp_hbm, ast_hbm, adt_hbm, src_hbm, dst_hbm, zn_hbm, zd_hbm,
               pnum_hbm, pden_hbm,
               acc_n, acc_d,
               sidx0, didx0, asr0, adr0, xrows0, wbuf0,
               sidx1, didx1, asr1, adr1, xrows1, wbuf1,
               sem_a0, sem_b0, sem_x0, sem_s0,
               sem_a1, sem_b1, sem_x1, sem_s1):
    c = lax.axis_index("c")
    s = lax.axis_index("s")
    wid = s * num_cores + c
    rows_per = NPAD // num_subcores
    r0 = s * rows_per
    pltpu.sync_copy(zn_hbm.at[pl.ds(r0, rows_per)], acc_n.at[pl.ds(r0, rows_per)])
    pltpu.sync_copy(zd_hbm.at[pl.ds(r0, rows_per)], acc_d.at[pl.ds(r0, rows_per)])
    plsc.subcore_barrier()

    bufs = (
        (sidx0, didx0, asr0, adr0, xrows0, wbuf0, sem_a0, sem_b0, sem_x0, sem_s0),
        (sidx1, didx1, asr1, adr1, xrows1, wbuf1, sem_a1, sem_b1, sem_x1, sem_s1),
    )

    def issue(g, buf):
        sidx, didx, asr, adr, xrows, _, sem_a, sem_b, sem_x, _ = buf
        pltpu.sync_copy(src_hbm.at[wid, g], sidx)
        pltpu.sync_copy(dst_hbm.at[wid, g], didx)
        pltpu.async_copy(ast_hbm.at[sidx], asr, sem_a)
        pltpu.async_copy(adt_hbm.at[didx], adr, sem_b)
        pltpu.async_copy(xp_hbm.at[sidx], xrows, sem_x)

    def wait_gathers(buf):
        sidx, didx, asr, adr, xrows, _, sem_a, sem_b, sem_x, _ = buf
        pltpu.make_async_copy(ast_hbm.at[sidx], asr, sem_a).wait()
        pltpu.make_async_copy(adt_hbm.at[didx], adr, sem_b).wait()
        pltpu.make_async_copy(xp_hbm.at[sidx], xrows, sem_x).wait()

    def compute_scatter(buf):
        _, didx, asr, adr, xrows, wbuf, _, _, _, sem_s = buf

        def edge_body(e, carry2):
            att = asr[e, :] + adr[e, :]
            att = jnp.where(att >= 0.0, att, att * NEG_SLOPE)
            wrow = jnp.exp(att)
            wbuf[e, :] = wrow
            for h in range(HEADS):
                wv = jnp.full((16,), wrow[h], jnp.float32)
                for j2 in range(2):
                    off = (h * 2 + j2) * 16
                    xrows[e, pl.ds(off, 16)] = xrows[e, pl.ds(off, 16)] * wv
            return carry2

        lax.fori_loop(0, K_EDGE, edge_body, 0, unroll=4)
        pltpu.async_copy(xrows, acc_n.at[didx], sem_s, add=True)
        pltpu.async_copy(wbuf, acc_d.at[didx], sem_s, add=True)
        pltpu.make_async_copy(xrows, acc_n.at[didx], sem_s).wait()
        pltpu.make_async_copy(wbuf, acc_d.at[didx], sem_s).wait()

    issue(0, bufs[0])

    def pair_body(g2, carry):
        a = 2 * g2
        issue(a + 1, bufs[1])
        wait_gathers(bufs[0])
        compute_scatter(bufs[0])
        issue(jnp.minimum(a + 2, num_rounds - 1), bufs[0])
        wait_gathers(bufs[1])
        compute_scatter(bufs[1])
        return carry

    lax.fori_loop(0, num_rounds // 2, pair_body, 0)
    wait_gathers(bufs[0])
    plsc.subcore_barrier()
    pltpu.sync_copy(acc_n.at[pl.ds(r0, rows_per)], pnum_hbm.at[c, pl.ds(r0, rows_per)])
    pltpu.sync_copy(acc_d.at[pl.ds(r0, rows_per)], pden_hbm.at[c, pl.ds(r0, rows_per)])


def _edge_pass(xp, ast_pad, adt_pad, src3, dst3, num_rounds, num_cores, num_subcores):
    mesh = plsc.VectorSubcoreMesh(core_axis_name="c", subcore_axis_name="s")
    zn = jnp.zeros((NPAD, HD), jnp.float32)
    zd = jnp.zeros((NPAD, 16), jnp.float32)
    body = functools.partial(_edge_body, num_rounds, num_cores, num_subcores)
    return pl.kernel(
        body,
        out_type=[
            jax.ShapeDtypeStruct((num_cores, NPAD, HD), jnp.float32),
            jax.ShapeDtypeStruct((num_cores, NPAD, 16), jnp.float32),
        ],
        mesh=mesh,
        compiler_params=pltpu.CompilerParams(use_tc_tiling_on_sc=False),
        scratch_types=(
            [
                pltpu.VMEM_SHARED((NPAD, HD), jnp.float32),
                pltpu.VMEM_SHARED((NPAD, 16), jnp.float32),
            ]
            + 2 * [
                pltpu.VMEM((K_EDGE,), jnp.int32),
                pltpu.VMEM((K_EDGE,), jnp.int32),
                pltpu.VMEM((K_EDGE, 16), jnp.float32),
                pltpu.VMEM((K_EDGE, 16), jnp.float32),
                pltpu.VMEM((K_EDGE, HD), jnp.float32),
                pltpu.VMEM((K_EDGE, 16), jnp.float32),
            ]
            + 8 * [pltpu.SemaphoreType.DMA]
        ),
    )(xp, ast_pad, adt_pad, src3, dst3, zn, zd)


def _combine_body(pnum_ref, pden_ref, bias_ref, out_ref):
    pn = pnum_ref[...]
    pd = pden_ref[...]
    num = pn[0] + pn[1]
    den = pd[0] + pd[1]
    r16 = lax.broadcasted_iota(jnp.int32, (16, HD), 0)
    c16 = lax.broadcasted_iota(jnp.int32, (16, HD), 1) // OUT_DIM
    m16 = jnp.where(r16 == c16, 1.0, 0.0)
    den_b = jnp.dot(den, m16, preferred_element_type=jnp.float32)
    out_ref[...] = num / den_b + bias_ref[...]


def _combine(pnum, pden, bias, num_cores):
    rows = 2000
    grid = N_NODES // rows
    return pl.pallas_call(
        _combine_body,
        grid=(grid,),
        in_specs=[
            pl.BlockSpec((num_cores, rows, HD), lambda i: (0, i, 0)),
            pl.BlockSpec((num_cores, rows, 16), lambda i: (0, i, 0)),
            pl.BlockSpec((1, HD), lambda i: (0, 0)),
        ],
        out_specs=pl.BlockSpec((rows, HD), lambda i: (i, 0)),
        out_shape=jax.ShapeDtypeStruct((N_NODES, HD), jnp.float32),
    )(pnum, pden, bias.reshape(1, HD))


def kernel(x, edge_index, W, att_src, att_dst, bias):
    num_cores, num_subcores = _sc_info()
    nw = num_cores * num_subcores
    xp, ast, adt = _project(x, W, att_src, att_dst)
    ast_pad = jnp.pad(ast, ((0, NPAD - N_NODES), (0, 0)))
    adt_pad = jnp.pad(adt, ((0, NPAD - N_NODES), (0, 0)))

    self_loops = jnp.arange(N_NODES, dtype=jnp.int32)
    src = jnp.concatenate([edge_index[0], self_loops])
    dst = jnp.concatenate([edge_index[1], self_loops])
    e_tot = src.shape[0]
    chunk = nw * K_EDGE
    num_rounds = 2 * -(-e_tot // (2 * chunk))
    e_pad = num_rounds * chunk - e_tot
    src = jnp.pad(src, (0, e_pad)).reshape(nw, num_rounds, K_EDGE)
    dst = jnp.pad(dst, (0, e_pad), constant_values=N_NODES).reshape(nw, num_rounds, K_EDGE)

    pnum, pden = _edge_pass(xp, ast_pad, adt_pad, src, dst, num_rounds, num_cores, num_subcores)
    return _combine(pnum, pden, bias, num_cores)


# speedup vs baseline: 91.2469x; 1.1555x over previous
import functools

import jax
import jax.numpy as jnp
from jax import lax
from jax.experimental import pallas as pl
from jax.experimental.pallas import tpu as pltpu
from jax.experimental.pallas import tpu_sc as plsc

N_NODES = 10000
IN_DIM = 128
HEADS = 4
OUT_DIM = 32
HD = HEADS * OUT_DIM
NEG_SLOPE = 0.2
NPAD = 10112
K_EDGE = 48


def _proj_body(x_ref, w_ref, asrc_ref, adst_ref, xp_ref, ast_ref, adt_ref):
    xp = jnp.dot(x_ref[...], w_ref[...], preferred_element_type=jnp.float32)
    xp_ref[...] = xp
    col = lax.broadcasted_iota(jnp.int32, (HD, 16), 0) // OUT_DIM
    k16 = lax.broadcasted_iota(jnp.int32, (HD, 16), 1)
    m2 = jnp.where((k16 < HEADS) & (col == k16), 1.0, 0.0)
    ast_ref[...] = jnp.dot(xp * asrc_ref[...], m2, preferred_element_type=jnp.float32)
    adt_ref[...] = jnp.dot(xp * adst_ref[...], m2, preferred_element_type=jnp.float32)


def _project(x, w, att_src, att_dst):
    rows = 2000
    grid = N_NODES // rows
    return pl.pallas_call(
        _proj_body,
        grid=(grid,),
        in_specs=[
            pl.BlockSpec((rows, IN_DIM), lambda i: (i, 0)),
            pl.BlockSpec((IN_DIM, HD), lambda i: (0, 0)),
            pl.BlockSpec((1, HD), lambda i: (0, 0)),
            pl.BlockSpec((1, HD), lambda i: (0, 0)),
        ],
        out_specs=[
            pl.BlockSpec((rows, HD), lambda i: (i, 0)),
            pl.BlockSpec((rows, 16), lambda i: (i, 0)),
            pl.BlockSpec((rows, 16), lambda i: (i, 0)),
        ],
        out_shape=[
            jax.ShapeDtypeStruct((N_NODES, HD), jnp.float32),
            jax.ShapeDtypeStruct((N_NODES, 16), jnp.float32),
            jax.ShapeDtypeStruct((N_NODES, 16), jnp.float32),
        ],
    )(x, w, att_src.reshape(1, HD), att_dst.reshape(1, HD))


def _sc_info():
    try:
        info = plsc.get_sparse_core_info()
        return info.num_cores, info.num_subcores
    except Exception:
        return 2, 16


def _edge_body(num_rounds, num_cores, num_subcores,
               xp_hbm, ast_hbm, adt_hbm, src_hbm, dst_hbm, zn_hbm, zd_hbm,
               pnum_hbm, pden_hbm,
               acc_n, acc_d, sidx_all, didx_all,
               asr0, adr0, xrows0, wbuf0,
               asr1, adr1, xrows1, wbuf1,
               sem_a0, sem_b0, sem_x0, sem_s0,
               sem_a1, sem_b1, sem_x1, sem_s1):
    c = lax.axis_index("c")
    s = lax.axis_index("s")
    wid = s * num_cores + c
    rows_per = NPAD // num_subcores
    r0 = s * rows_per
    pltpu.sync_copy(zn_hbm.at[pl.ds(r0, rows_per)], acc_n.at[pl.ds(r0, rows_per)])
    pltpu.sync_copy(zd_hbm.at[pl.ds(r0, rows_per)], acc_d.at[pl.ds(r0, rows_per)])
    pltpu.sync_copy(src_hbm.at[wid], sidx_all)
    pltpu.sync_copy(dst_hbm.at[wid], didx_all)
    plsc.subcore_barrier()

    bufs = (
        (asr0, adr0, xrows0, wbuf0, sem_a0, sem_b0, sem_x0, sem_s0),
        (asr1, adr1, xrows1, wbuf1, sem_a1, sem_b1, sem_x1, sem_s1),
    )

    def issue(g, buf):
        asr, adr, xrows, _, sem_a, sem_b, sem_x, _ = buf
        pltpu.async_copy(ast_hbm.at[sidx_all.at[g]], asr, sem_a)
        pltpu.async_copy(adt_hbm.at[didx_all.at[g]], adr, sem_b)
        pltpu.async_copy(xp_hbm.at[sidx_all.at[g]], xrows, sem_x)

    def wait_gathers(g, buf):
        asr, adr, xrows, _, sem_a, sem_b, sem_x, _ = buf
        pltpu.make_async_copy(ast_hbm.at[sidx_all.at[g]], asr, sem_a).wait()
        pltpu.make_async_copy(adt_hbm.at[didx_all.at[g]], adr, sem_b).wait()
        pltpu.make_async_copy(xp_hbm.at[sidx_all.at[g]], xrows, sem_x).wait()

    def compute_scatter(g, buf):
        asr, adr, xrows, wbuf, _, _, _, sem_s = buf
        didx = didx_all.at[g]

        def edge_body(e, carry2):
            att = asr[e, :] + adr[e, :]
            att = jnp.where(att >= 0.0, att, att * NEG_SLOPE)
            wrow = jnp.exp(att)
            wbuf[e, :] = wrow
            for h in range(HEADS):
                wv = jnp.full((16,), wrow[h], jnp.float32)
                for j2 in range(2):
                    off = (h * 2 + j2) * 16
                    xrows[e, pl.ds(off, 16)] = xrows[e, pl.ds(off, 16)] * wv
            return carry2

        lax.fori_loop(0, K_EDGE, edge_body, 0, unroll=8)
        pltpu.async_copy(xrows, acc_n.at[didx], sem_s, add=True)
        pltpu.async_copy(wbuf, acc_d.at[didx], sem_s, add=True)
        pltpu.make_async_copy(xrows, acc_n.at[didx], sem_s).wait()
        pltpu.make_async_copy(wbuf, acc_d.at[didx], sem_s).wait()

    issue(0, bufs[0])

    def pair_body(g2, carry):
        a = 2 * g2
        issue(a + 1, bufs[1])
        wait_gathers(a, bufs[0])
        compute_scatter(a, bufs[0])
        issue(jnp.minimum(a + 2, num_rounds - 1), bufs[0])
        wait_gathers(a + 1, bufs[1])
        compute_scatter(a + 1, bufs[1])
        return carry

    lax.fori_loop(0, num_rounds // 2, pair_body, 0)
    wait_gathers(num_rounds - 1, bufs[0])
    plsc.subcore_barrier()
    pltpu.sync_copy(acc_n.at[pl.ds(r0, rows_per)], pnum_hbm.at[c, pl.ds(r0, rows_per)])
    pltpu.sync_copy(acc_d.at[pl.ds(r0, rows_per)], pden_hbm.at[c, pl.ds(r0, rows_per)])


def _edge_pass(xp, ast_pad, adt_pad, src3, dst3, num_rounds, num_cores, num_subcores):
    mesh = plsc.VectorSubcoreMesh(core_axis_name="c", subcore_axis_name="s")
    zn = jnp.zeros((NPAD, HD), jnp.float32)
    zd = jnp.zeros((NPAD, 16), jnp.float32)
    body = functools.partial(_edge_body, num_rounds, num_cores, num_subcores)
    return pl.kernel(
        body,
        out_type=[
            jax.ShapeDtypeStruct((num_cores, NPAD, HD), jnp.float32),
            jax.ShapeDtypeStruct((num_cores, NPAD, 16), jnp.float32),
        ],
        mesh=mesh,
        compiler_params=pltpu.CompilerParams(use_tc_tiling_on_sc=False),
        scratch_types=(
            [
                pltpu.VMEM_SHARED((NPAD, HD), jnp.float32),
                pltpu.VMEM_SHARED((NPAD, 16), jnp.float32),
            ]
            + [
                pltpu.VMEM((num_rounds, K_EDGE), jnp.int32),
                pltpu.VMEM((num_rounds, K_EDGE), jnp.int32),
            ]
            + 2 * [
                pltpu.VMEM((K_EDGE, 16), jnp.float32),
                pltpu.VMEM((K_EDGE, 16), jnp.float32),
                pltpu.VMEM((K_EDGE, HD), jnp.float32),
                pltpu.VMEM((K_EDGE, 16), jnp.float32),
            ]
            + 8 * [pltpu.SemaphoreType.DMA]
        ),
    )(xp, ast_pad, adt_pad, src3, dst3, zn, zd)


def _combine_body(pnum_ref, pden_ref, bias_ref, out_ref):
    pn = pnum_ref[...]
    pd = pden_ref[...]
    num = pn[0] + pn[1]
    den = pd[0] + pd[1]
    r16 = lax.broadcasted_iota(jnp.int32, (16, HD), 0)
    c16 = lax.broadcasted_iota(jnp.int32, (16, HD), 1) // OUT_DIM
    m16 = jnp.where(r16 == c16, 1.0, 0.0)
    den_b = jnp.dot(den, m16, preferred_element_type=jnp.float32)
    out_ref[...] = num / den_b + bias_ref[...]


def _combine(pnum, pden, bias, num_cores):
    rows = 2000
    grid = N_NODES // rows
    return pl.pallas_call(
        _combine_body,
        grid=(grid,),
        in_specs=[
            pl.BlockSpec((num_cores, rows, HD), lambda i: (0, i, 0)),
            pl.BlockSpec((num_cores, rows, 16), lambda i: (0, i, 0)),
            pl.BlockSpec((1, HD), lambda i: (0, 0)),
        ],
        out_specs=pl.BlockSpec((rows, HD), lambda i: (i, 0)),
        out_shape=jax.ShapeDtypeStruct((N_NODES, HD), jnp.float32),
    )(pnum, pden, bias.reshape(1, HD))


def kernel(x, edge_index, W, att_src, att_dst, bias):
    num_cores, num_subcores = _sc_info()
    nw = num_cores * num_subcores
    xp, ast, adt = _project(x, W, att_src, att_dst)
    ast_pad = jnp.pad(ast, ((0, NPAD - N_NODES), (0, 0)))
    adt_pad = jnp.pad(adt, ((0, NPAD - N_NODES), (0, 0)))

    self_loops = jnp.arange(N_NODES, dtype=jnp.int32)
    src = jnp.concatenate([edge_index[0], self_loops])
    dst = jnp.concatenate([edge_index[1], self_loops])
    e_tot = src.shape[0]
    chunk = nw * K_EDGE
    num_rounds = 2 * -(-e_tot // (2 * chunk))
    e_pad = num_rounds * chunk - e_tot
    src = jnp.pad(src, (0, e_pad)).reshape(nw, num_rounds, K_EDGE)
    dst = jnp.pad(dst, (0, e_pad), constant_values=N_NODES).reshape(nw, num_rounds, K_EDGE)

    pnum, pden = _edge_pass(xp, ast_pad, adt_pad, src, dst, num_rounds, num_cores, num_subcores)
    return _combine(pnum, pden, bias, num_cores)


# speedup vs baseline: 125.1244x; 1.3713x over previous
import functools

import jax
import jax.numpy as jnp
from jax import lax
from jax.experimental import pallas as pl
from jax.experimental.pallas import tpu as pltpu
from jax.experimental.pallas import tpu_sc as plsc

N_NODES = 10000
IN_DIM = 128
HEADS = 4
OUT_DIM = 32
HD = HEADS * OUT_DIM
NEG_SLOPE = 0.2
NPAD = 10112
K_EDGE = 48


def _proj_body(x_ref, w_ref, asrc_ref, adst_ref, xp_ref, ast_ref, adt_ref):
    xp = jnp.dot(x_ref[...], w_ref[...], preferred_element_type=jnp.float32)
    xp_ref[...] = xp
    col = lax.broadcasted_iota(jnp.int32, (HD, 16), 0) // OUT_DIM
    k16 = lax.broadcasted_iota(jnp.int32, (HD, 16), 1)
    m2 = jnp.where((k16 < HEADS) & (col == k16), 1.0, 0.0)
    ast_ref[...] = jnp.dot(xp * asrc_ref[...], m2, preferred_element_type=jnp.float32)
    adt_ref[...] = jnp.dot(xp * adst_ref[...], m2, preferred_element_type=jnp.float32)


def _project(x, w, att_src, att_dst):
    rows = 2000
    grid = N_NODES // rows
    return pl.pallas_call(
        _proj_body,
        grid=(grid,),
        in_specs=[
            pl.BlockSpec((rows, IN_DIM), lambda i: (i, 0)),
            pl.BlockSpec((IN_DIM, HD), lambda i: (0, 0)),
            pl.BlockSpec((1, HD), lambda i: (0, 0)),
            pl.BlockSpec((1, HD), lambda i: (0, 0)),
        ],
        out_specs=[
            pl.BlockSpec((rows, HD), lambda i: (i, 0)),
            pl.BlockSpec((rows, 16), lambda i: (i, 0)),
            pl.BlockSpec((rows, 16), lambda i: (i, 0)),
        ],
        out_shape=[
            jax.ShapeDtypeStruct((N_NODES, HD), jnp.float32),
            jax.ShapeDtypeStruct((N_NODES, 16), jnp.float32),
            jax.ShapeDtypeStruct((N_NODES, 16), jnp.float32),
        ],
    )(x, w, att_src.reshape(1, HD), att_dst.reshape(1, HD))


def _sc_info():
    try:
        info = plsc.get_sparse_core_info()
        return info.num_cores, info.num_subcores
    except Exception:
        return 2, 16


def _edge_body(num_rounds, num_cores, num_subcores,
               xp_hbm, ast_hbm, adt_hbm, src_hbm, dst_hbm, zn_hbm, zd_hbm,
               pnum_hbm, pden_hbm,
               acc_n, acc_d, sidx_all, didx_all,
               asr0, adr0, xrows0, wbuf0,
               asr1, adr1, xrows1, wbuf1,
               sem_a0, sem_b0, sem_x0, sem_s0,
               sem_a1, sem_b1, sem_x1, sem_s1):
    c = lax.axis_index("c")
    s = lax.axis_index("s")
    wid = s * num_cores + c
    rows_per = NPAD // num_subcores
    r0 = s * rows_per
    pltpu.sync_copy(zn_hbm.at[pl.ds(r0, rows_per)], acc_n.at[pl.ds(r0, rows_per)])
    pltpu.sync_copy(zd_hbm.at[pl.ds(r0, rows_per)], acc_d.at[pl.ds(r0, rows_per)])
    pltpu.sync_copy(src_hbm.at[wid], sidx_all)
    pltpu.sync_copy(dst_hbm.at[wid], didx_all)
    plsc.subcore_barrier()

    bufs = (
        (asr0, adr0, xrows0, wbuf0, sem_a0, sem_b0, sem_x0, sem_s0),
        (asr1, adr1, xrows1, wbuf1, sem_a1, sem_b1, sem_x1, sem_s1),
    )

    def issue(g, buf):
        asr, adr, xrows, _, sem_a, sem_b, sem_x, _ = buf
        pltpu.async_copy(ast_hbm.at[sidx_all.at[g]], asr, sem_a)
        pltpu.async_copy(adt_hbm.at[didx_all.at[g]], adr, sem_b)
        pltpu.async_copy(xp_hbm.at[sidx_all.at[g]], xrows, sem_x)

    def wait_gathers(g, buf):
        asr, adr, xrows, _, sem_a, sem_b, sem_x, _ = buf
        pltpu.make_async_copy(ast_hbm.at[sidx_all.at[g]], asr, sem_a).wait()
        pltpu.make_async_copy(adt_hbm.at[didx_all.at[g]], adr, sem_b).wait()
        pltpu.make_async_copy(xp_hbm.at[sidx_all.at[g]], xrows, sem_x).wait()

    def compute_scatter(g, buf):
        asr, adr, xrows, wbuf, _, _, _, sem_s = buf
        didx = didx_all.at[g]

        def w_block(b, carry2):
            e0 = b * 8
            atts = [asr[e0 + i, :] + adr[e0 + i, :] for i in range(8)]
            atts = [jnp.where(a >= 0.0, a, a * NEG_SLOPE) for a in atts]
            ws = [jnp.exp(a) for a in atts]
            for i in range(8):
                wbuf[e0 + i, :] = ws[i]
            return carry2

        lax.fori_loop(0, K_EDGE // 8, w_block, 0)

        def mul_block(b, carry2):
            e0 = b * 2
            wrows = [wbuf[e0 + i, :] for i in range(2)]
            xvs = [
                [xrows[e0 + i, pl.ds(j * 16, 16)] for j in range(8)]
                for i in range(2)
            ]
            outs = []
            for i in range(2):
                wvs = [jnp.full((16,), wrows[i][h], jnp.float32) for h in range(HEADS)]
                outs.append([xvs[i][j] * wvs[j // 2] for j in range(8)])
            for i in range(2):
                for j in range(8):
                    xrows[e0 + i, pl.ds(j * 16, 16)] = outs[i][j]
            return carry2

        lax.fori_loop(0, K_EDGE // 2, mul_block, 0, unroll=2)
        pltpu.async_copy(xrows, acc_n.at[didx], sem_s, add=True)
        pltpu.async_copy(wbuf, acc_d.at[didx], sem_s, add=True)
        pltpu.make_async_copy(xrows, acc_n.at[didx], sem_s).wait()
        pltpu.make_async_copy(wbuf, acc_d.at[didx], sem_s).wait()

    issue(0, bufs[0])

    def pair_body(g2, carry):
        a = 2 * g2
        issue(a + 1, bufs[1])
        wait_gathers(a, bufs[0])
        compute_scatter(a, bufs[0])
        issue(jnp.minimum(a + 2, num_rounds - 1), bufs[0])
        wait_gathers(a + 1, bufs[1])
        compute_scatter(a + 1, bufs[1])
        return carry

    lax.fori_loop(0, num_rounds // 2, pair_body, 0)
    wait_gathers(num_rounds - 1, bufs[0])
    plsc.subcore_barrier()
    pltpu.sync_copy(acc_n.at[pl.ds(r0, rows_per)], pnum_hbm.at[c, pl.ds(r0, rows_per)])
    pltpu.sync_copy(acc_d.at[pl.ds(r0, rows_per)], pden_hbm.at[c, pl.ds(r0, rows_per)])


def _edge_pass(xp, ast_pad, adt_pad, src3, dst3, num_rounds, num_cores, num_subcores):
    mesh = plsc.VectorSubcoreMesh(core_axis_name="c", subcore_axis_name="s")
    zn = jnp.zeros((NPAD, HD), jnp.float32)
    zd = jnp.zeros((NPAD, 16), jnp.float32)
    body = functools.partial(_edge_body, num_rounds, num_cores, num_subcores)
    return pl.kernel(
        body,
        out_type=[
            jax.ShapeDtypeStruct((num_cores, NPAD, HD), jnp.float32),
            jax.ShapeDtypeStruct((num_cores, NPAD, 16), jnp.float32),
        ],
        mesh=mesh,
        compiler_params=pltpu.CompilerParams(use_tc_tiling_on_sc=False),
        scratch_types=(
            [
                pltpu.VMEM_SHARED((NPAD, HD), jnp.float32),
                pltpu.VMEM_SHARED((NPAD, 16), jnp.float32),
            ]
            + [
                pltpu.VMEM((num_rounds, K_EDGE), jnp.int32),
                pltpu.VMEM((num_rounds, K_EDGE), jnp.int32),
            ]
            + 2 * [
                pltpu.VMEM((K_EDGE, 16), jnp.float32),
                pltpu.VMEM((K_EDGE, 16), jnp.float32),
                pltpu.VMEM((K_EDGE, HD), jnp.float32),
                pltpu.VMEM((K_EDGE, 16), jnp.float32),
            ]
            + 8 * [pltpu.SemaphoreType.DMA]
        ),
    )(xp, ast_pad, adt_pad, src3, dst3, zn, zd)


def _combine_body(pnum_ref, pden_ref, bias_ref, out_ref):
    pn = pnum_ref[...]
    pd = pden_ref[...]
    num = pn[0] + pn[1]
    den = pd[0] + pd[1]
    r16 = lax.broadcasted_iota(jnp.int32, (16, HD), 0)
    c16 = lax.broadcasted_iota(jnp.int32, (16, HD), 1) // OUT_DIM
    m16 = jnp.where(r16 == c16, 1.0, 0.0)
    den_b = jnp.dot(den, m16, preferred_element_type=jnp.float32)
    out_ref[...] = num / den_b + bias_ref[...]


def _combine(pnum, pden, bias, num_cores):
    rows = 2000
    grid = N_NODES // rows
    return pl.pallas_call(
        _combine_body,
        grid=(grid,),
        in_specs=[
            pl.BlockSpec((num_cores, rows, HD), lambda i: (0, i, 0)),
            pl.BlockSpec((num_cores, rows, 16), lambda i: (0, i, 0)),
            pl.BlockSpec((1, HD), lambda i: (0, 0)),
        ],
        out_specs=pl.BlockSpec((rows, HD), lambda i: (i, 0)),
        out_shape=jax.ShapeDtypeStruct((N_NODES, HD), jnp.float32),
    )(pnum, pden, bias.reshape(1, HD))


def kernel(x, edge_index, W, att_src, att_dst, bias):
    num_cores, num_subcores = _sc_info()
    nw = num_cores * num_subcores
    xp, ast, adt = _project(x, W, att_src, att_dst)
    ast_pad = jnp.pad(ast, ((0, NPAD - N_NODES), (0, 0)))
    adt_pad = jnp.pad(adt, ((0, NPAD - N_NODES), (0, 0)))

    self_loops = jnp.arange(N_NODES, dtype=jnp.int32)
    src = jnp.concatenate([edge_index[0], self_loops])
    dst = jnp.concatenate([edge_index[1], self_loops])
    e_tot = src.shape[0]
    chunk = nw * K_EDGE
    num_rounds = 2 * -(-e_tot // (2 * chunk))
    e_pad = num_rounds * chunk - e_tot
    src = jnp.pad(src, (0, e_pad)).reshape(nw, num_rounds, K_EDGE)
    dst = jnp.pad(dst, (0, e_pad), constant_values=N_NODES).reshape(nw, num_rounds, K_EDGE)

    pnum, pden = _edge_pass(xp, ast_pad, adt_pad, src, dst, num_rounds, num_cores, num_subcores)
    return _combine(pnum, pden, bias, num_cores)
